# baseline (speedup 1.0000x reference)
import functools

import jax
import jax.numpy as jnp
from jax import lax
from jax.experimental import pallas as pl
from jax.experimental.pallas import tpu as pltpu
from jax.experimental.pallas import tpu_sc as plsc

N = 10000
E = 320000
NP = 10240
IN_DIM = 128
HID = 256

EB = 40
K = 5
RPW = NP // 16

ROW_BLK = 2048


def _mesh():
    return plsc.VectorSubcoreMesh(core_axis_name="c", subcore_axis_name="s")


def _sc_degree(ei_flat, zeros1):
    rows_pw = E // 32 // EB
    nchunks = rows_pw // K

    @functools.partial(
        pl.kernel,
        out_type=jax.ShapeDtypeStruct((2, NP), jnp.float32),
        mesh=_mesh(),
        scratch_types=[
            pltpu.VMEM((2, K, EB), jnp.int32),
            pltpu.VMEM((EB,), jnp.float32),
            pltpu.VMEM_SHARED((NP,), jnp.float32),
            pltpu.SemaphoreType.DMA,
            pltpu.SemaphoreType.DMA,
        ],
    )
    def deg_kernel(dst_hbm, z_hbm, out_hbm, di_v, ones_v, acc_sh, isem, ssem):
        c = lax.axis_index("c")
        s = lax.axis_index("s")
        pltpu.sync_copy(z_hbm.at[pl.ds(s * RPW, RPW)], acc_sh.at[pl.ds(s * RPW, RPW)])

        @pl.loop(0, EB, step=16)
        def _(i):
            ones_v[pl.ds(i, 16)] = jnp.full((16,), 1.0, dtype=jnp.float32)

        plsc.subcore_barrier()
        base = (c * 16 + s) * rows_pw

        def issue_idx(chunk, p):
            for k in range(K):
                pltpu.async_copy(dst_hbm.at[pl.ds(E + (base + chunk * K + k) * EB, EB)],
                                 di_v.at[p, k], isem)

        def drain_idx(p):
            for k in range(K):
                pltpu.make_async_copy(dst_hbm.at[pl.ds(0, EB)], di_v.at[p, k],
                                      isem).wait()

        issue_idx(0, 0)

        @pl.loop(0, nchunks, step=2)
        def _(cc):
            for p in (0, 1):
                cur = cc + p
                drain_idx(p)
                ss = [pltpu.async_copy(ones_v, acc_sh.at[di_v.at[p, k]], ssem,
                                       add=True) for k in range(K)]

                @pl.when(cur + 1 < nchunks)
                def _():
                    issue_idx(cur + 1, 1 - p)

                for h in ss:
                    h.wait()

        plsc.subcore_barrier()
        pltpu.sync_copy(acc_sh.at[pl.ds(s * RPW, RPW)], out_hbm.at[c, pl.ds(s * RPW, RPW)])

    return deg_kernel(ei_flat, zeros1)


def _sc_propagate(ei_flat, tab0, tab1, zeros2, edges_split_by_core,
                  do_scatter=True, eb=EB, kc=4):
    nwork = 32 if edges_split_by_core else 16
    KC = kc
    EBL = eb
    rows_pw = E // nwork // EBL
    nchunks = rows_pw // KC
    rem = rows_pw - nchunks * KC
    ntail = (nchunks - 2) % 4
    main_end = nchunks - ntail

    @functools.partial(
        pl.kernel,
        out_type=jax.ShapeDtypeStruct((2, NP, IN_DIM), jnp.float32),
        mesh=_mesh(),
        scratch_types=[
            pltpu.VMEM((4, KC, EBL), jnp.int32),
            pltpu.VMEM((4, KC, EBL), jnp.int32),
            pltpu.VMEM((2, KC, EBL, IN_DIM), jnp.float32),
            pltpu.VMEM_SHARED((NP, IN_DIM), jnp.float32),
            pltpu.SemaphoreType.DMA,
            pltpu.SemaphoreType.DMA,
            pltpu.SemaphoreType.DMA,
            pltpu.SemaphoreType.DMA,
            pltpu.SemaphoreType.DMA,
        ],
    )
    def prop_kernel(ei_hbm, t0_hbm, t1_hbm, z_hbm, out_hbm,
                    si_v, di_v, rows_v, acc_sh, isem0, isem1, gsem, ssem0, ssem1):
        c = lax.axis_index("c")
        s = lax.axis_index("s")
        isems = (isem0, isem1)
        ssems = (ssem0, ssem1)
        pltpu.sync_copy(z_hbm.at[pl.ds(s * RPW, RPW)], acc_sh.at[pl.ds(s * RPW, RPW)])
        plsc.subcore_barrier()

        if edges_split_by_core:
            base = (c * 16 + s) * rows_pw
        else:
            base = s * rows_pw

        def issue_idx(chunk, q):
            for k in range(KC):
                off = (base + chunk * KC + k) * EBL
                pltpu.async_copy(ei_hbm.at[pl.ds(off, EBL)], si_v.at[q, k],
                                 isems[q % 2])
                pltpu.async_copy(ei_hbm.at[pl.ds(E + off, EBL)], di_v.at[q, k],
                                 isems[q % 2])

        def drain_idx(q):
            for k in range(KC):
                pltpu.make_async_copy(ei_hbm.at[pl.ds(0, EBL)], si_v.at[q, k],
                                      isems[q % 2]).wait()
                pltpu.make_async_copy(ei_hbm.at[pl.ds(0, EBL)], di_v.at[q, k],
                                      isems[q % 2]).wait()

        def run(tab):
            def chunk(cur, p, q, first, issue_mode):
                if not first and do_scatter:
                    for k in range(KC):
                        pltpu.make_async_copy(rows_v.at[p, k],
                                              acc_sh.at[di_v.at[q, k]],
                                              ssems[p]).wait()
                drain_idx(q)
                gs = [pltpu.async_copy(tab.at[si_v.at[q, k]], rows_v.at[p, k], gsem)
                      for k in range(KC)]
                if issue_mode == "always":
                    issue_idx(cur + 2, (q + 2) % 4)
                elif issue_mode == "when":
                    @pl.when(cur + 2 < nchunks)
                    def _():
                        issue_idx(cur + 2, (q + 2) % 4)
                for g in gs:
                    g.wait()
                if do_scatter:
                    for k in range(KC):
                        pltpu.async_copy(rows_v.at[p, k], acc_sh.at[di_v.at[q, k]],
                                         ssems[p], add=True)

            issue_idx(0, 0)
            issue_idx(1, 1)
            chunk(0, 0, 0, True, "always")
            chunk(1, 1, 1, True, "always")

            @pl.loop(2, main_end, step=4)
            def _(cc):
                for i in range(4):
                    chunk(cc + i, i % 2, (2 + i) % 4, False, "when")

            for t in range(main_end, nchunks):
                chunk(t, t % 2, t % 4, False,
                      "always" if t + 2 < nchunks else "none")

            if do_scatter:
                for t in (nchunks - 2, nchunks - 1):
                    for k in range(KC):
                        pltpu.make_async_copy(rows_v.at[t % 2, k],
                                              acc_sh.at[di_v.at[t % 4, k]],
                                              ssems[t % 2]).wait()

            for r in range(rem):
                roff = (base + nchunks * KC + r) * EBL
                pltpu.sync_copy(ei_hbm.at[pl.ds(roff, EBL)], si_v.at[0, r])
                pltpu.sync_copy(ei_hbm.at[pl.ds(E + roff, EBL)], di_v.at[0, r])
                pltpu.sync_copy(tab.at[si_v.at[0, r]], rows_v.at[0, r])
                if do_scatter:
                    pltpu.sync_copy(rows_v.at[0, r], acc_sh.at[di_v.at[0, r]],
                                    add=True)

        @pl.when(c == 0)
        def _():
            run(t0_hbm)

        @pl.when(c == 1)
        def _():
            run(t1_hbm)

        plsc.subcore_barrier()
        pltpu.sync_copy(acc_sh.at[pl.ds(s * RPW, RPW)], out_hbm.at[c, pl.ds(s * RPW, RPW)])

    return prop_kernel(ei_flat, tab0, tab1, zeros2)


def _dot(a, b):
    dims = (((1,), (0,)), ((), ()))

    def d(u, v):
        return lax.dot_general(u, v, dims, preferred_element_type=jnp.float32)

    a_hi = a.astype(jnp.bfloat16)
    a_lo = (a - a_hi.astype(jnp.float32)).astype(jnp.bfloat16)
    b_hi = b.astype(jnp.bfloat16)
    b_lo = (b - b_hi.astype(jnp.float32)).astype(jnp.bfloat16)
    return d(a_hi, b_hi) + d(a_hi, b_lo) + d(a_lo, b_hi)


def _tc_prescale(deg2, xp):
    def body(deg_ref, x_ref, dinv_ref, dinv2_ref, xs_ref):
        d = deg_ref[0] + deg_ref[1] + 1.0
        dv = lax.rsqrt(d)
        dinv_ref[...] = dv
        dinv2_ref[...] = dv * dv
        xs_ref[...] = x_ref[...] * dv[:, None]

    return pl.pallas_call(
        body,
        grid=(NP // ROW_BLK,),
        in_specs=[
            pl.BlockSpec((2, ROW_BLK), lambda i: (0, i)),
            pl.BlockSpec((ROW_BLK, IN_DIM), lambda i: (i, 0)),
        ],
        out_specs=[
            pl.BlockSpec((ROW_BLK,), lambda i: (i,)),
            pl.BlockSpec((ROW_BLK,), lambda i: (i,)),
            pl.BlockSpec((ROW_BLK, IN_DIM), lambda i: (i, 0)),
        ],
        out_shape=[
            jax.ShapeDtypeStruct((NP,), jnp.float32),
            jax.ShapeDtypeStruct((NP,), jnp.float32),
            jax.ShapeDtypeStruct((NP, IN_DIM), jnp.float32),
        ],
    )(deg2, xp)


def _tc_layer1(p13, xp, dinv, dinv2, W1, b1r):
    def body(p_ref, x_ref, dv_ref, dv2_ref, w_ref, b_ref, h1_ref, ha_ref, hb_ref):
        p = p_ref[0] + p_ref[1]
        dv = dv_ref[...][:, None]
        g = dv * p + dv2_ref[...][:, None] * x_ref[...]
        z = _dot(g, w_ref[...]) + b_ref[...]
        h1 = jnp.maximum(z, 0.0)
        h1_ref[...] = h1
        hs = h1 * dv
        ha_ref[...] = hs[:, :IN_DIM]
        hb_ref[...] = hs[:, IN_DIM:]

    return pl.pallas_call(
        body,
        grid=(NP // ROW_BLK,),
        in_specs=[
            pl.BlockSpec((2, ROW_BLK, IN_DIM), lambda i: (0, i, 0)),
            pl.BlockSpec((ROW_BLK, IN_DIM), lambda i: (i, 0)),
            pl.BlockSpec((ROW_BLK,), lambda i: (i,)),
            pl.BlockSpec((ROW_BLK,), lambda i: (i,)),
            pl.BlockSpec((IN_DIM, HID), lambda i: (0, 0)),
            pl.BlockSpec((1, HID), lambda i: (0, 0)),
        ],
        out_specs=[
            pl.BlockSpec((ROW_BLK, HID), lambda i: (i, 0)),
            pl.BlockSpec((ROW_BLK, IN_DIM), lambda i: (i, 0)),
            pl.BlockSpec((ROW_BLK, IN_DIM), lambda i: (i, 0)),
        ],
        out_shape=[
            jax.ShapeDtypeStruct((NP, HID), jnp.float32),
            jax.ShapeDtypeStruct((NP, IN_DIM), jnp.float32),
            jax.ShapeDtypeStruct((NP, IN_DIM), jnp.float32),
        ],
    )(p13, xp, dinv, dinv2, W1, b1r)


def _tc_layer2(p23, h1, dinv, dinv2, W2, b2r, Wl, blr):
    def body(p_ref, h1_ref, dv_ref, dv2_ref, w2_ref, b2_ref, wl_ref, bl_ref, out_ref):
        h1v = h1_ref[...]
        dv = dv_ref[...][:, None]
        dv2 = dv2_ref[...][:, None]
        ga = dv * p_ref[0] + dv2 * h1v[:, :IN_DIM]
        gb = dv * p_ref[1] + dv2 * h1v[:, IN_DIM:]
        z = _dot(ga, w2_ref[:IN_DIM, :]) + _dot(gb, w2_ref[IN_DIM:, :]) + b2_ref[...]
        o2 = jnp.maximum(z, 0.0)
        out_ref[...] = jnp.sum(o2 * wl_ref[...], axis=1) + bl_ref[0, 0]

    return pl.pallas_call(
        body,
        grid=(NP // ROW_BLK,),
        in_specs=[
            pl.BlockSpec((2, ROW_BLK, IN_DIM), lambda i: (0, i, 0)),
            pl.BlockSpec((ROW_BLK, HID), lambda i: (i, 0)),
            pl.BlockSpec((ROW_BLK,), lambda i: (i,)),
            pl.BlockSpec((ROW_BLK,), lambda i: (i,)),
            pl.BlockSpec((HID, HID), lambda i: (0, 0)),
            pl.BlockSpec((1, HID), lambda i: (0, 0)),
            pl.BlockSpec((1, HID), lambda i: (0, 0)),
            pl.BlockSpec((1, 1), lambda i: (0, 0)),
        ],
        out_specs=[pl.BlockSpec((ROW_BLK,), lambda i: (i,))],
        out_shape=[jax.ShapeDtypeStruct((NP,), jnp.float32)],
    )(p23, h1, dinv, dinv2, W2, b2r, Wl, blr)


def kernel(x, edge_index, W1, b1, W2, b2, Wl, bl):
    ei_flat = edge_index.reshape(2 * E)
    xp = jnp.pad(x, ((0, NP - N), (0, 0)))
    zeros1 = jnp.zeros((NP,), jnp.float32)
    zeros2 = jnp.zeros((NP, IN_DIM), jnp.float32)

    deg2 = _sc_degree(ei_flat, zeros1)
    dinv, dinv2, xs = _tc_prescale(deg2, xp)
    p1 = _sc_propagate(ei_flat, xs, xs, zeros2, True)
    h1, ha, hb = _tc_layer1(p1, xp, dinv, dinv2, W1, b1.reshape(1, HID))
    p2 = _sc_propagate(ei_flat, ha, hb, zeros2, False, eb=16, kc=5)
    (logits,) = _tc_layer2(p2, h1, dinv, dinv2, W2, b2.reshape(1, HID),
                           Wl.reshape(1, HID), bl.reshape(1, 1))
    return logits[:N]


# speedup vs baseline: 1.1941x; 1.1941x over previous
import functools

import jax
import jax.numpy as jnp
from jax import lax
from jax.experimental import pallas as pl
from jax.experimental.pallas import tpu as pltpu
from jax.experimental.pallas import tpu_sc as plsc

N = 10000
E = 320000
NP = 10240
IN_DIM = 128
HID = 256

EB = 40
K = 5
RPW = NP // 16

ROW_BLK = 2048


def _mesh():
    return plsc.VectorSubcoreMesh(core_axis_name="c", subcore_axis_name="s")


def _sc_degree(ei_flat, zeros1):
    rows_pw = E // 32 // EB
    nchunks = rows_pw // K

    @functools.partial(
        pl.kernel,
        out_type=jax.ShapeDtypeStruct((2, NP), jnp.float32),
        mesh=_mesh(),
        scratch_types=[
            pltpu.VMEM((2, K, EB), jnp.int32),
            pltpu.VMEM((EB,), jnp.float32),
            pltpu.VMEM_SHARED((NP,), jnp.float32),
            pltpu.SemaphoreType.DMA,
            pltpu.SemaphoreType.DMA,
        ],
    )
    def deg_kernel(dst_hbm, z_hbm, out_hbm, di_v, ones_v, acc_sh, isem, ssem):
        c = lax.axis_index("c")
        s = lax.axis_index("s")
        pltpu.sync_copy(z_hbm.at[pl.ds(s * RPW, RPW)], acc_sh.at[pl.ds(s * RPW, RPW)])

        @pl.loop(0, EB, step=16)
        def _(i):
            ones_v[pl.ds(i, 16)] = jnp.full((16,), 1.0, dtype=jnp.float32)

        plsc.subcore_barrier()
        base = (c * 16 + s) * rows_pw

        def issue_idx(chunk, p):
            for k in range(K):
                pltpu.async_copy(dst_hbm.at[pl.ds(E + (base + chunk * K + k) * EB, EB)],
                                 di_v.at[p, k], isem)

        def drain_idx(p):
            for k in range(K):
                pltpu.make_async_copy(dst_hbm.at[pl.ds(0, EB)], di_v.at[p, k],
                                      isem).wait()

        issue_idx(0, 0)

        @pl.loop(0, nchunks, step=2)
        def _(cc):
            for p in (0, 1):
                cur = cc + p
                drain_idx(p)
                ss = [pltpu.async_copy(ones_v, acc_sh.at[di_v.at[p, k]], ssem,
                                       add=True) for k in range(K)]

                @pl.when(cur + 1 < nchunks)
                def _():
                    issue_idx(cur + 1, 1 - p)

                for h in ss:
                    h.wait()

        plsc.subcore_barrier()
        pltpu.sync_copy(acc_sh.at[pl.ds(s * RPW, RPW)], out_hbm.at[c, pl.ds(s * RPW, RPW)])

    return deg_kernel(ei_flat, zeros1)


def _sc_propagate(ei_flat, tab0, tab1, zeros2, edges_split_by_core,
                  do_scatter=True, eb=EB, kc=4):
    nwork = 32 if edges_split_by_core else 16
    KC = kc
    EBL = eb
    rows_pw = E // nwork // EBL
    nchunks = rows_pw // KC
    rem = rows_pw - nchunks * KC
    ntail = (nchunks - 2) % 4
    main_end = nchunks - ntail

    @functools.partial(
        pl.kernel,
        out_type=jax.ShapeDtypeStruct((2, NP, IN_DIM), jnp.float32),
        mesh=_mesh(),
        scratch_types=[
            pltpu.VMEM((4, KC, EBL), jnp.int32),
            pltpu.VMEM((4, KC, EBL), jnp.int32),
            pltpu.VMEM((2, KC, EBL, IN_DIM), jnp.float32),
            pltpu.VMEM_SHARED((NP, IN_DIM), jnp.float32),
            pltpu.SemaphoreType.DMA,
            pltpu.SemaphoreType.DMA,
            pltpu.SemaphoreType.DMA,
            pltpu.SemaphoreType.DMA,
            pltpu.SemaphoreType.DMA,
        ],
    )
    def prop_kernel(ei_hbm, t0_hbm, t1_hbm, z_hbm, out_hbm,
                    si_v, di_v, rows_v, acc_sh, isem0, isem1, gsem, ssem0, ssem1):
        c = lax.axis_index("c")
        s = lax.axis_index("s")
        isems = (isem0, isem1)
        ssems = (ssem0, ssem1)
        pltpu.sync_copy(z_hbm.at[pl.ds(s * RPW, RPW)], acc_sh.at[pl.ds(s * RPW, RPW)])
        plsc.subcore_barrier()

        if edges_split_by_core:
            base = (c * 16 + s) * rows_pw
        else:
            base = s * rows_pw

        def issue_idx(chunk, q):
            for k in range(KC):
                off = (base + chunk * KC + k) * EBL
                pltpu.async_copy(ei_hbm.at[pl.ds(off, EBL)], si_v.at[q, k],
                                 isems[q % 2])
                pltpu.async_copy(ei_hbm.at[pl.ds(E + off, EBL)], di_v.at[q, k],
                                 isems[q % 2])

        def drain_idx(q):
            for k in range(KC):
                pltpu.make_async_copy(ei_hbm.at[pl.ds(0, EBL)], si_v.at[q, k],
                                      isems[q % 2]).wait()
                pltpu.make_async_copy(ei_hbm.at[pl.ds(0, EBL)], di_v.at[q, k],
                                      isems[q % 2]).wait()

        def run(tab):
            def chunk(cur, p, q, first, issue_mode):
                if not first and do_scatter:
                    for k in range(KC):
                        pltpu.make_async_copy(rows_v.at[p, k],
                                              acc_sh.at[di_v.at[q, k]],
                                              ssems[p]).wait()
                drain_idx(q)
                gs = [pltpu.async_copy(tab.at[si_v.at[q, k]], rows_v.at[p, k], gsem)
                      for k in range(KC)]
                if issue_mode == "always":
                    issue_idx(cur + 2, (q + 2) % 4)
                elif issue_mode == "when":
                    @pl.when(cur + 2 < nchunks)
                    def _():
                        issue_idx(cur + 2, (q + 2) % 4)
                for g in gs:
                    g.wait()
                if do_scatter:
                    for k in range(KC):
                        pltpu.async_copy(rows_v.at[p, k], acc_sh.at[di_v.at[q, k]],
                                         ssems[p], add=True)

            issue_idx(0, 0)
            issue_idx(1, 1)
            chunk(0, 0, 0, True, "always")
            chunk(1, 1, 1, True, "always")

            @pl.loop(2, main_end, step=4)
            def _(cc):
                for i in range(4):
                    chunk(cc + i, i % 2, (2 + i) % 4, False, "when")

            for t in range(main_end, nchunks):
                chunk(t, t % 2, t % 4, False,
                      "always" if t + 2 < nchunks else "none")

            if do_scatter:
                for t in (nchunks - 2, nchunks - 1):
                    for k in range(KC):
                        pltpu.make_async_copy(rows_v.at[t % 2, k],
                                              acc_sh.at[di_v.at[t % 4, k]],
                                              ssems[t % 2]).wait()

            for r in range(rem):
                roff = (base + nchunks * KC + r) * EBL
                pltpu.sync_copy(ei_hbm.at[pl.ds(roff, EBL)], si_v.at[0, r])
                pltpu.sync_copy(ei_hbm.at[pl.ds(E + roff, EBL)], di_v.at[0, r])
                pltpu.sync_copy(tab.at[si_v.at[0, r]], rows_v.at[0, r])
                if do_scatter:
                    pltpu.sync_copy(rows_v.at[0, r], acc_sh.at[di_v.at[0, r]],
                                    add=True)

        @pl.when(c == 0)
        def _():
            run(t0_hbm)

        @pl.when(c == 1)
        def _():
            run(t1_hbm)

        plsc.subcore_barrier()
        pltpu.sync_copy(acc_sh.at[pl.ds(s * RPW, RPW)], out_hbm.at[c, pl.ds(s * RPW, RPW)])

    return prop_kernel(ei_flat, tab0, tab1, zeros2)


def _dot(a, b):
    dims = (((1,), (0,)), ((), ()))

    def d(u, v):
        return lax.dot_general(u, v, dims, preferred_element_type=jnp.float32)

    a_hi = a.astype(jnp.bfloat16)
    a_lo = (a - a_hi.astype(jnp.float32)).astype(jnp.bfloat16)
    b_hi = b.astype(jnp.bfloat16)
    b_lo = (b - b_hi.astype(jnp.float32)).astype(jnp.bfloat16)
    return d(a_hi, b_hi) + d(a_hi, b_lo) + d(a_lo, b_hi)


def _tc_prescale(deg2, xp):
    def body(deg_ref, x_ref, dinv_ref, dinv2_ref, xs_ref):
        d = deg_ref[0] + deg_ref[1] + 1.0
        dv = lax.rsqrt(d)
        dinv_ref[...] = dv
        dinv2_ref[...] = dv * dv
        xs_ref[...] = x_ref[...] * dv[:, None]

    return pl.pallas_call(
        body,
        grid=(NP // ROW_BLK,),
        in_specs=[
            pl.BlockSpec((2, ROW_BLK), lambda i: (0, i)),
            pl.BlockSpec((ROW_BLK, IN_DIM), lambda i: (i, 0)),
        ],
        out_specs=[
            pl.BlockSpec((ROW_BLK,), lambda i: (i,)),
            pl.BlockSpec((ROW_BLK,), lambda i: (i,)),
            pl.BlockSpec((ROW_BLK, IN_DIM), lambda i: (i, 0)),
        ],
        out_shape=[
            jax.ShapeDtypeStruct((NP,), jnp.float32),
            jax.ShapeDtypeStruct((NP,), jnp.float32),
            jax.ShapeDtypeStruct((NP, IN_DIM), jnp.float32),
        ],
    )(deg2, xp)


def _tc_layer1(p13, xp, dinv, dinv2, W1, b1r):
    def body(p_ref, x_ref, dv_ref, dv2_ref, w_ref, b_ref, h1_ref, ha_ref, hb_ref):
        p = p_ref[0] + p_ref[1]
        dv = dv_ref[...][:, None]
        g = dv * p + dv2_ref[...][:, None] * x_ref[...]
        z = _dot(g, w_ref[...]) + b_ref[...]
        h1 = jnp.maximum(z, 0.0)
        h1_ref[...] = h1
        hs = h1 * dv
        ha_ref[...] = hs[:, :IN_DIM]
        hb_ref[...] = hs[:, IN_DIM:]

    return pl.pallas_call(
        body,
        grid=(NP // ROW_BLK,),
        in_specs=[
            pl.BlockSpec((2, ROW_BLK, IN_DIM), lambda i: (0, i, 0)),
            pl.BlockSpec((ROW_BLK, IN_DIM), lambda i: (i, 0)),
            pl.BlockSpec((ROW_BLK,), lambda i: (i,)),
            pl.BlockSpec((ROW_BLK,), lambda i: (i,)),
            pl.BlockSpec((IN_DIM, HID), lambda i: (0, 0)),
            pl.BlockSpec((1, HID), lambda i: (0, 0)),
        ],
        out_specs=[
            pl.BlockSpec((ROW_BLK, HID), lambda i: (i, 0)),
            pl.BlockSpec((ROW_BLK, IN_DIM), lambda i: (i, 0)),
            pl.BlockSpec((ROW_BLK, IN_DIM), lambda i: (i, 0)),
        ],
        out_shape=[
            jax.ShapeDtypeStruct((NP, HID), jnp.float32),
            jax.ShapeDtypeStruct((NP, IN_DIM), jnp.float32),
            jax.ShapeDtypeStruct((NP, IN_DIM), jnp.float32),
        ],
    )(p13, xp, dinv, dinv2, W1, b1r)


def _tc_layer2(p23, h1, dinv, dinv2, W2, b2r, Wl, blr):
    def body(p_ref, h1_ref, dv_ref, dv2_ref, w2_ref, b2_ref, wl_ref, bl_ref, out_ref):
        h1v = h1_ref[...]
        dv = dv_ref[...][:, None]
        dv2 = dv2_ref[...][:, None]
        ga = dv * p_ref[0] + dv2 * h1v[:, :IN_DIM]
        gb = dv * p_ref[1] + dv2 * h1v[:, IN_DIM:]
        z = _dot(ga, w2_ref[:IN_DIM, :]) + _dot(gb, w2_ref[IN_DIM:, :]) + b2_ref[...]
        o2 = jnp.maximum(z, 0.0)
        out_ref[...] = jnp.sum(o2 * wl_ref[...], axis=1) + bl_ref[0, 0]

    return pl.pallas_call(
        body,
        grid=(NP // ROW_BLK,),
        in_specs=[
            pl.BlockSpec((2, ROW_BLK, IN_DIM), lambda i: (0, i, 0)),
            pl.BlockSpec((ROW_BLK, HID), lambda i: (i, 0)),
            pl.BlockSpec((ROW_BLK,), lambda i: (i,)),
            pl.BlockSpec((ROW_BLK,), lambda i: (i,)),
            pl.BlockSpec((HID, HID), lambda i: (0, 0)),
            pl.BlockSpec((1, HID), lambda i: (0, 0)),
            pl.BlockSpec((1, HID), lambda i: (0, 0)),
            pl.BlockSpec((1, 1), lambda i: (0, 0)),
        ],
        out_specs=[pl.BlockSpec((ROW_BLK,), lambda i: (i,))],
        out_shape=[jax.ShapeDtypeStruct((NP,), jnp.float32)],
    )(p23, h1, dinv, dinv2, W2, b2r, Wl, blr)


def kernel(x, edge_index, W1, b1, W2, b2, Wl, bl):
    ei_flat = edge_index.reshape(2 * E)
    xp = jnp.pad(x, ((0, NP - N), (0, 0)))
    zeros1 = jnp.zeros((NP,), jnp.float32)
    zeros2 = jnp.zeros((NP, IN_DIM), jnp.float32)

    deg2 = _sc_degree(ei_flat, zeros1)
    dinv, dinv2, xs = _tc_prescale(deg2, xp)
    p1 = _sc_propagate(ei_flat, xs, xs, zeros2, True, eb=80, kc=2)
    h1, ha, hb = _tc_layer1(p1, xp, dinv, dinv2, W1, b1.reshape(1, HID))
    p2 = _sc_propagate(ei_flat, ha, hb, zeros2, False, eb=80, kc=2)
    (logits,) = _tc_layer2(p2, h1, dinv, dinv2, W2, b2.reshape(1, HID),
                           Wl.reshape(1, HID), bl.reshape(1, 1))
    return logits[:N]
